# baseline (speedup 1.0000x reference)
import functools

import jax
import jax.numpy as jnp
from jax import lax
from jax.experimental import pallas as pl
from jax.experimental.pallas import tpu as pltpu
from jax.experimental.pallas import tpu_sc as plsc

E = 320000
D = 128
N = 10000

NC = 2
NS = 16
NW = NC * NS

EPW = E // NW
C = 128
NCH = EPW // C
TAIL = EPW - NCH * C

N_PAD = NS * 640
WB = 632
WB_LAST = N - (NS - 1) * WB


def _sc_body(msg_hbm, idx_hbm, out_hbm,
             idx_a, idx_b, idx_t, rows_a, rows_b, rows_t, acc,
             sem_a, sem_b, sem_t):
  c = lax.axis_index("c")
  s = lax.axis_index("s")
  wid = c * NS + s
  base = wid * EPW

  def _start(j, idx_v, rows_v, sem):
    off = base + j * C
    pltpu.async_copy(idx_hbm.at[pl.ds(off, C)], idx_v, sem)
    pltpu.async_copy(msg_hbm.at[pl.ds(off, C)], rows_v, sem)

  def _finish(j, idx_v, rows_v, sem):
    pltpu.make_async_copy(idx_hbm.at[pl.ds(base, C)], idx_v, sem).wait()
    pltpu.make_async_copy(msg_hbm.at[pl.ds(base, C)], rows_v, sem).wait()
    pltpu.sync_copy(rows_v, acc.at[idx_v], add=True)

  zvec = jnp.zeros((16,), jnp.float32)

  def _zero_row(i, _):
    for k in range(D // 16):
      rows_a[i, pl.ds(k * 16, 16)] = zvec
    return 0

  lax.fori_loop(0, C, _zero_row, 0)
  zd = [pltpu.async_copy(rows_a, acc.at[pl.ds(s * (N_PAD // NS) + k * C, C)],
                         sem_t)
        for k in range(N_PAD // NS // C)]
  for d in zd:
    d.wait()

  _start(0, idx_a, rows_a, sem_a)
  _start(1, idx_b, rows_b, sem_b)
  toff = base + NCH * C
  pltpu.async_copy(idx_hbm.at[pl.ds(toff, TAIL)], idx_t, sem_t)
  pltpu.async_copy(msg_hbm.at[pl.ds(toff, TAIL)], rows_t, sem_t)
  plsc.subcore_barrier()

  def _outer(g, _):
    for b, (idx_v, rows_v, sem) in enumerate(
        ((idx_a, rows_a, sem_a), (idx_b, rows_b, sem_b))):
      j = 2 * g + b
      _finish(j, idx_v, rows_v, sem)

      @pl.when(j + 2 < NCH)
      def _():
        _start(j + 2, idx_v, rows_v, sem)
    return 0

  lax.fori_loop(0, NCH // 2, _outer, 0)

  pltpu.make_async_copy(idx_hbm.at[pl.ds(toff, TAIL)], idx_t, sem_t).wait()
  pltpu.make_async_copy(msg_hbm.at[pl.ds(toff, TAIL)], rows_t, sem_t).wait()
  pltpu.sync_copy(rows_t, acc.at[idx_t], add=True)

  plsc.subcore_barrier()

  r0 = s * WB

  @pl.when(s < NS - 1)
  def _():
    pltpu.sync_copy(acc.at[pl.ds(r0, WB)], out_hbm.at[c, pl.ds(r0, WB)])

  @pl.when(s == NS - 1)
  def _():
    pltpu.sync_copy(acc.at[pl.ds((NS - 1) * WB, WB_LAST)],
                    out_hbm.at[c, pl.ds((NS - 1) * WB, WB_LAST)])


@jax.jit
def _sc_partials(msg, idx):
  mesh = plsc.VectorSubcoreMesh(core_axis_name="c", subcore_axis_name="s",
                                num_cores=NC, num_subcores=NS)
  return pl.kernel(
      _sc_body,
      out_type=jax.ShapeDtypeStruct((NC, N, D), jnp.float32),
      mesh=mesh,
      scratch_types=[
          pltpu.VMEM((C,), jnp.int32),
          pltpu.VMEM((C,), jnp.int32),
          pltpu.VMEM((TAIL,), jnp.int32),
          pltpu.VMEM((C, D), jnp.float32),
          pltpu.VMEM((C, D), jnp.float32),
          pltpu.VMEM((TAIL, D), jnp.float32),
          pltpu.VMEM_SHARED((N_PAD, D), jnp.float32),
          pltpu.SemaphoreType.DMA,
          pltpu.SemaphoreType.DMA,
          pltpu.SemaphoreType.DMA,
      ],
  )(msg, idx)


def _combine_body(p_ref, o_ref):
  o_ref[...] = p_ref[0] + p_ref[1]


@jax.jit
def _tc_combine(partials):
  bn = 10000
  return pl.pallas_call(
      _combine_body,
      out_shape=jax.ShapeDtypeStruct((N, D), jnp.float32),
      grid=(N // bn,),
      in_specs=[pl.BlockSpec((NC, bn, D), lambda i: (0, i, 0))],
      out_specs=pl.BlockSpec((bn, D), lambda i: (i, 0)),
  )(partials)


def kernel(msg, index, t, dim_size):
  del t, dim_size
  idx = index.astype(jnp.int32)
  return _tc_combine(_sc_partials(msg, idx))


# speedup vs baseline: 1.0176x; 1.0176x over previous
import functools

import jax
import jax.numpy as jnp
from jax import lax
from jax.experimental import pallas as pl
from jax.experimental.pallas import tpu as pltpu
from jax.experimental.pallas import tpu_sc as plsc

E = 320000
D = 128
N = 10000

NC = 2
NS = 16
NW = NC * NS

EPW = E // NW
C = 128
NCH = EPW // C
TAIL = EPW - NCH * C

N_PAD = NS * 640
WB = 632
WB_LAST = N - (NS - 1) * WB


def _sc_body(msg_hbm, idx_hbm, out_hbm,
             idx_a, idx_b, idx_t, rows_a, rows_b, rows_t, acc,
             sem_a, sem_b, sem_t, sem_z):
  c = lax.axis_index("c")
  s = lax.axis_index("s")
  wid = c * NS + s
  base = wid * EPW

  def _start(j, idx_v, rows_v, sem):
    off = base + j * C
    pltpu.async_copy(idx_hbm.at[pl.ds(off, C)], idx_v, sem)
    pltpu.async_copy(msg_hbm.at[pl.ds(off, C)], rows_v, sem)

  def _finish(j, idx_v, rows_v, sem):
    pltpu.make_async_copy(idx_hbm.at[pl.ds(base, C)], idx_v, sem).wait()
    pltpu.make_async_copy(msg_hbm.at[pl.ds(base, C)], rows_v, sem).wait()
    pltpu.sync_copy(rows_v, acc.at[idx_v], add=True)

  zvec = jnp.zeros((16,), jnp.float32)

  def _zero_row(i, _):
    for k in range(D // 16):
      rows_b[i, pl.ds(k * 16, 16)] = zvec
    return 0

  _start(0, idx_a, rows_a, sem_a)
  toff = base + NCH * C
  pltpu.async_copy(idx_hbm.at[pl.ds(toff, TAIL)], idx_t, sem_t)
  pltpu.async_copy(msg_hbm.at[pl.ds(toff, TAIL)], rows_t, sem_t)
  lax.fori_loop(0, C, _zero_row, 0)
  zd = [pltpu.async_copy(rows_b, acc.at[pl.ds(s * (N_PAD // NS) + k * C, C)],
                         sem_z)
        for k in range(N_PAD // NS // C)]
  for d in zd:
    d.wait()

  _start(1, idx_b, rows_b, sem_b)
  plsc.subcore_barrier()

  def _outer(g, _):
    for b, (idx_v, rows_v, sem) in enumerate(
        ((idx_a, rows_a, sem_a), (idx_b, rows_b, sem_b))):
      j = 2 * g + b
      _finish(j, idx_v, rows_v, sem)

      @pl.when(j + 2 < NCH)
      def _():
        _start(j + 2, idx_v, rows_v, sem)
    return 0

  lax.fori_loop(0, NCH // 2, _outer, 0)

  pltpu.make_async_copy(idx_hbm.at[pl.ds(toff, TAIL)], idx_t, sem_t).wait()
  pltpu.make_async_copy(msg_hbm.at[pl.ds(toff, TAIL)], rows_t, sem_t).wait()
  pltpu.sync_copy(rows_t, acc.at[idx_t], add=True)

  plsc.subcore_barrier()

  r0 = s * WB

  @pl.when(s < NS - 1)
  def _():
    pltpu.sync_copy(acc.at[pl.ds(r0, WB)], out_hbm.at[c, pl.ds(r0, WB)])

  @pl.when(s == NS - 1)
  def _():
    pltpu.sync_copy(acc.at[pl.ds((NS - 1) * WB, WB_LAST)],
                    out_hbm.at[c, pl.ds((NS - 1) * WB, WB_LAST)])


@jax.jit
def _sc_partials(msg, idx):
  mesh = plsc.VectorSubcoreMesh(core_axis_name="c", subcore_axis_name="s",
                                num_cores=NC, num_subcores=NS)
  return pl.kernel(
      _sc_body,
      out_type=jax.ShapeDtypeStruct((NC, N, D), jnp.float32),
      mesh=mesh,
      scratch_types=[
          pltpu.VMEM((C,), jnp.int32),
          pltpu.VMEM((C,), jnp.int32),
          pltpu.VMEM((TAIL,), jnp.int32),
          pltpu.VMEM((C, D), jnp.float32),
          pltpu.VMEM((C, D), jnp.float32),
          pltpu.VMEM((TAIL, D), jnp.float32),
          pltpu.VMEM_SHARED((N_PAD, D), jnp.float32),
          pltpu.SemaphoreType.DMA,
          pltpu.SemaphoreType.DMA,
          pltpu.SemaphoreType.DMA,
          pltpu.SemaphoreType.DMA,
      ],
  )(msg, idx)


def _combine_body(p_ref, o_ref):
  o_ref[...] = p_ref[0] + p_ref[1]


@jax.jit
def _tc_combine(partials):
  bn = 5000
  return pl.pallas_call(
      _combine_body,
      out_shape=jax.ShapeDtypeStruct((N, D), jnp.float32),
      grid=(N // bn,),
      in_specs=[pl.BlockSpec((NC, bn, D), lambda i: (0, i, 0))],
      out_specs=pl.BlockSpec((bn, D), lambda i: (i, 0)),
  )(partials)


def kernel(msg, index, t, dim_size):
  del t, dim_size
  idx = index.astype(jnp.int32)
  return _tc_combine(_sc_partials(msg, idx))


# speedup vs baseline: 1.0192x; 1.0017x over previous
import jax
import jax.numpy as jnp
from jax import lax
from jax.experimental import pallas as pl
from jax.experimental.pallas import tpu as pltpu
from jax.experimental.pallas import tpu_sc as plsc

E = 320000
D = 128
N = 10000

NC = 2
NS = 16
NW = NC * NS

EPW = E // NW
C = 128
NCH = EPW // C
TAIL = EPW - NCH * C

N_PAD = NS * 640
WB = 632
WB_LAST = N - (NS - 1) * WB


def _sc_body(msg_hbm, idx_hbm, out_hbm,
             idx_a, idx_b, idx_t, rows_a, rows_b, rows_t, acc,
             sem_a, sem_b, sem_t, sem_z):
  c = lax.axis_index("c")
  s = lax.axis_index("s")
  wid = c * NS + s
  base = wid * EPW

  def _start(j, idx_v, rows_v, sem):
    off = base + j * C
    pltpu.async_copy(idx_hbm.at[pl.ds(off, C)], idx_v, sem)
    pltpu.async_copy(msg_hbm.at[pl.ds(off, C)], rows_v, sem)

  def _finish(j, idx_v, rows_v, sem):
    pltpu.make_async_copy(idx_hbm.at[pl.ds(base, C)], idx_v, sem).wait()
    pltpu.make_async_copy(msg_hbm.at[pl.ds(base, C)], rows_v, sem).wait()
    pltpu.sync_copy(rows_v, acc.at[idx_v], add=True)

  zvec = jnp.zeros((16,), jnp.float32)

  def _zero_row(i, _):
    for k in range(D // 16):
      rows_b[i, pl.ds(k * 16, 16)] = zvec
    return 0

  _start(0, idx_a, rows_a, sem_a)
  toff = base + NCH * C
  pltpu.async_copy(idx_hbm.at[pl.ds(toff, TAIL)], idx_t, sem_t)
  pltpu.async_copy(msg_hbm.at[pl.ds(toff, TAIL)], rows_t, sem_t)
  lax.fori_loop(0, C, _zero_row, 0)
  zd = [pltpu.async_copy(rows_b, acc.at[pl.ds(s * (N_PAD // NS) + k * C, C)],
                         sem_z)
        for k in range(N_PAD // NS // C)]
  for d in zd:
    d.wait()

  _start(1, idx_b, rows_b, sem_b)
  plsc.subcore_barrier()

  def _outer(g, _):
    for b, (idx_v, rows_v, sem) in enumerate(
        ((idx_a, rows_a, sem_a), (idx_b, rows_b, sem_b))):
      j = 2 * g + b
      _finish(j, idx_v, rows_v, sem)

      @pl.when(j + 2 < NCH)
      def _():
        _start(j + 2, idx_v, rows_v, sem)
    return 0

  lax.fori_loop(0, NCH // 2, _outer, 0)

  pltpu.make_async_copy(idx_hbm.at[pl.ds(toff, TAIL)], idx_t, sem_t).wait()
  pltpu.make_async_copy(msg_hbm.at[pl.ds(toff, TAIL)], rows_t, sem_t).wait()
  pltpu.sync_copy(rows_t, acc.at[idx_t], add=True)

  plsc.subcore_barrier()

  r0 = s * WB

  @pl.when(s < NS - 1)
  def _():
    pltpu.sync_copy(acc.at[pl.ds(r0, WB)], out_hbm.at[c, pl.ds(r0, WB)])

  @pl.when(s == NS - 1)
  def _():
    pltpu.sync_copy(acc.at[pl.ds((NS - 1) * WB, WB_LAST)],
                    out_hbm.at[c, pl.ds((NS - 1) * WB, WB_LAST)])


@jax.jit
def _sc_partials(msg, idx):
  mesh = plsc.VectorSubcoreMesh(core_axis_name="c", subcore_axis_name="s",
                                num_cores=NC, num_subcores=NS)
  return pl.kernel(
      _sc_body,
      out_type=jax.ShapeDtypeStruct((NC, N, D), jnp.float32),
      mesh=mesh,
      scratch_types=[
          pltpu.VMEM((C,), jnp.int32),
          pltpu.VMEM((C,), jnp.int32),
          pltpu.VMEM((TAIL,), jnp.int32),
          pltpu.VMEM((C, D), jnp.float32),
          pltpu.VMEM((C, D), jnp.float32),
          pltpu.VMEM((TAIL, D), jnp.float32),
          pltpu.VMEM_SHARED((N_PAD, D), jnp.float32),
          pltpu.SemaphoreType.DMA,
          pltpu.SemaphoreType.DMA,
          pltpu.SemaphoreType.DMA,
          pltpu.SemaphoreType.DMA,
      ],
  )(msg, idx)


def _combine_body(p_ref, o_ref):
  o_ref[...] = p_ref[0] + p_ref[1]


@jax.jit
def _tc_combine(partials):
  bn = 5000
  return pl.pallas_call(
      _combine_body,
      out_shape=jax.ShapeDtypeStruct((N, D), jnp.float32),
      grid=(N // bn,),
      in_specs=[pl.BlockSpec((NC, bn, D), lambda i: (0, i, 0))],
      out_specs=pl.BlockSpec((bn, D), lambda i: (i, 0)),
  )(partials)


def kernel(msg, index, t, dim_size):
  del t, dim_size
  idx = index.astype(jnp.int32)
  return _tc_combine(_sc_partials(msg, idx))
